# baseline (speedup 1.0000x reference)
import functools

import jax
import jax.numpy as jnp
from jax import lax
from jax.experimental import pallas as pl
from jax.experimental.pallas import tpu as pltpu
from jax.experimental.pallas import tpu_sc as plsc

N_NODES = 10000
DIM_IN = 128
DIM_H = 256
DIM_OUT = 64

NC = 2
NS = 16
NW = NC * NS
L = 16

N_PAD = 10240
ROWS_PER_TILE = N_PAD // NS
NQ = 4
QC = DIM_H // NQ
CB = 128

KSUP = 16
ESUP = KSUP * CB
NBUF = 4
GAHEAD = 3
XR = 32
HG = ROWS_PER_TILE // L


def _ring(nblk, s, hs_sp, agg_sp, src_hbm, dst_hbm, src_buf, dst_buf,
          rows_v, gsem, ssem):
    def idx_ref(buf, k):
        return buf.at[(k // KSUP) % 2, k % KSUP]

    def issue_gather(kg, bg):
        pltpu.async_copy(hs_sp.at[idx_ref(src_buf, kg)], rows_v.at[bg],
                         gsem[bg])

    for k0 in range(GAHEAD):
        issue_gather(k0, k0)

    def chunk(k, b):
        pltpu.make_async_copy(hs_sp.at[idx_ref(src_buf, k)],
                              rows_v.at[b], gsem[b]).wait()
        pltpu.async_copy(rows_v.at[b], agg_sp.at[idx_ref(dst_buf, k)],
                         ssem[b], add=True)
        kg = k + GAHEAD
        bg = (b + GAHEAD) % NBUF

        @pl.when(kg < nblk)
        def _():
            @pl.when((kg % KSUP == 0) & (kg // KSUP > 0))
            def _():
                pltpu.sync_copy(src_hbm.at[s, kg // KSUP],
                                src_buf.at[(kg // KSUP) % 2])
                pltpu.sync_copy(dst_hbm.at[s, kg // KSUP],
                                dst_buf.at[(kg // KSUP) % 2])

            @pl.when(kg >= NBUF)
            def _():
                pltpu.make_async_copy(
                    rows_v.at[bg],
                    agg_sp.at[idx_ref(dst_buf, kg - NBUF)],
                    ssem[bg]).wait()

            issue_gather(kg, bg)

    def group(g, carry):
        for b in range(NBUF):
            chunk(NBUF * g + b, b)
        return carry

    lax.fori_loop(0, nblk // NBUF, group, 0)
    for d in range(NBUF):
        k = nblk - NBUF + d
        pltpu.make_async_copy(rows_v.at[k % NBUF],
                              agg_sp.at[idx_ref(dst_buf, k)],
                              ssem[k % NBUF]).wait()


def _hist_body(eh, dst_hbm, out_hbm, dst_v, hist_v):
    c = lax.axis_index("c")
    s = lax.axis_index("s")
    wid = s * NC + c
    pltpu.sync_copy(dst_hbm.at[wid], dst_v)
    zeros16 = jnp.zeros((L,), jnp.float32)

    def zbody(g, carry):
        hist_v[pl.ds(g * L, L)] = zeros16
        return carry

    lax.fori_loop(0, N_PAD // L, zbody, 0)
    ones16 = jnp.ones((L,), jnp.float32)

    def body(g, carry):
        idx = dst_v[pl.ds(g * L, L)]
        plsc.addupdate_scatter(hist_v, [idx], ones16)
        return carry

    lax.fori_loop(0, eh // L, body, 0)
    pltpu.sync_copy(hist_v, out_hbm.at[wid])


def _make_hist(eh):
    return pl.kernel(
        functools.partial(_hist_body, eh),
        out_type=jax.ShapeDtypeStruct((NW, N_PAD), jnp.float32),
        mesh=plsc.VectorSubcoreMesh(core_axis_name="c", subcore_axis_name="s"),
        compiler_params=pltpu.CompilerParams(needs_layout_passes=False),
        scratch_types=[
            pltpu.VMEM((eh,), jnp.int32),
            pltpu.VMEM((N_PAD,), jnp.float32),
        ],
    )


def _mp_body(nsup, nqpc, hs_hbm, src_hbm, dst_hbm, out_hbm,
             src_buf, dst_buf, rows_v, hs_sp, agg_sp, *sems):
    gsem = sems[:NBUF]
    ssem = sems[NBUF:]
    c = lax.axis_index("c")
    s = lax.axis_index("s")
    r0 = s * ROWS_PER_TILE
    nblk = nsup * KSUP

    for p in range(nqpc):
        q = c * nqpc + p
        seeds = [
            (hs_hbm.at[q, pl.ds(r0, ROWS_PER_TILE)],
             hs_sp.at[pl.ds(r0, ROWS_PER_TILE)], gsem[0]),
            (hs_hbm.at[q, pl.ds(r0, ROWS_PER_TILE)],
             agg_sp.at[pl.ds(r0, ROWS_PER_TILE)], gsem[1]),
            (src_hbm.at[s, 0], src_buf.at[0], ssem[0]),
            (dst_hbm.at[s, 0], dst_buf.at[0], ssem[1]),
        ]
        for sref, dref, sem in seeds:
            pltpu.async_copy(sref, dref, sem)
        for sref, dref, sem in seeds:
            pltpu.make_async_copy(sref, dref, sem).wait()
        plsc.subcore_barrier()
        _ring(nblk, s, hs_sp, agg_sp, src_hbm, dst_hbm, src_buf, dst_buf,
              rows_v, gsem, ssem)
        plsc.subcore_barrier()
        pltpu.sync_copy(agg_sp.at[pl.ds(r0, ROWS_PER_TILE)],
                        out_hbm.at[q, pl.ds(r0, ROWS_PER_TILE)])


def _make_mp(nsup, nqpc):
    return pl.kernel(
        functools.partial(_mp_body, nsup, nqpc),
        out_type=jax.ShapeDtypeStruct((NC * nqpc, N_PAD, QC), jnp.float32),
        mesh=plsc.VectorSubcoreMesh(core_axis_name="c", subcore_axis_name="s"),
        compiler_params=pltpu.CompilerParams(needs_layout_passes=False,
                                             use_tc_tiling_on_sc=False),
        scratch_types=[
            pltpu.VMEM((2, KSUP, CB), jnp.int32),
            pltpu.VMEM((2, KSUP, CB), jnp.int32),
            pltpu.VMEM((NBUF, CB, QC), jnp.float32),
            pltpu.VMEM_SHARED((N_PAD, QC), jnp.float32),
            pltpu.VMEM_SHARED((N_PAD, QC), jnp.float32),
        ] + [pltpu.SemaphoreType.DMA] * (2 * NBUF),
    )


def _dinv_from(deg_ref):
    dsum = jnp.sum(deg_ref[...], axis=0) + 1.0
    return lax.rsqrt(dsum)[:, None]


def _scale_tc(x_ref, deg_ref, out_ref):
    dinv = _dinv_from(deg_ref)
    xs = x_ref[...] * dinv
    out_ref[0] = xs[:, :QC]
    out_ref[1] = xs[:, QC:]


def _store_quarters(out_ref, hs):
    for i in range(NQ):
        out_ref[i] = hs[:, i * QC:(i + 1) * QC]


def _mid_tc(aggx_ref, w1_ref, b1_ref, w2_ref, deg_ref, out_ref):
    dinv = _dinv_from(deg_ref)
    al = aggx_ref[0] * dinv
    ar = aggx_ref[1] * dinv
    h1 = (jnp.dot(al, w1_ref[:QC, :], preferred_element_type=jnp.float32)
          + jnp.dot(ar, w1_ref[QC:, :], preferred_element_type=jnp.float32))
    o1 = jnp.maximum(h1 + b1_ref[...], 0.0)
    h2 = jnp.dot(o1, w2_ref[...], preferred_element_type=jnp.float32)
    _store_quarters(out_ref, h2 * dinv)


def _relu_quarters(agg_ref, b_ref, dinv):
    return [jnp.maximum(agg_ref[i] * dinv + b_ref[:, i * QC:(i + 1) * QC], 0.0)
            for i in range(NQ)]


def _out_tc(agg_ref, b2_ref, w3_ref, b3_ref, deg_ref, out_ref):
    dinv = _dinv_from(deg_ref)
    hq = _relu_quarters(agg_ref, b2_ref, dinv)
    logits = sum(jnp.dot(hq[i], w3_ref[i * QC:(i + 1) * QC, :],
                         preferred_element_type=jnp.float32)
                 for i in range(NQ)) + b3_ref[...]
    m = jnp.max(logits, axis=1, keepdims=True)
    sh = logits - m
    lse = jnp.log(jnp.sum(jnp.exp(sh), axis=1, keepdims=True))
    out_ref[...] = sh - lse


BN = 1024


def _scale_call(xp, deg_parts):
    return pl.pallas_call(
        _scale_tc,
        grid=(N_PAD // BN,),
        in_specs=[
            pl.BlockSpec((BN, DIM_IN), lambda i: (i, 0)),
            pl.BlockSpec((NW, BN), lambda i: (0, i)),
        ],
        out_specs=pl.BlockSpec((NC, BN, QC), lambda i: (0, i, 0)),
        out_shape=jax.ShapeDtypeStruct((NC, N_PAD, QC), jnp.float32),
    )(xp, deg_parts)


def _mid_call(aggx, w1, b1r, w2, deg_parts):
    return pl.pallas_call(
        _mid_tc,
        grid=(N_PAD // BN,),
        in_specs=[
            pl.BlockSpec((NC, BN, QC), lambda i: (0, i, 0)),
            pl.BlockSpec((DIM_IN, DIM_H), lambda i: (0, 0)),
            pl.BlockSpec((1, DIM_H), lambda i: (0, 0)),
            pl.BlockSpec((DIM_H, DIM_H), lambda i: (0, 0)),
            pl.BlockSpec((NW, BN), lambda i: (0, i)),
        ],
        out_specs=pl.BlockSpec((NQ, BN, QC), lambda i: (0, i, 0)),
        out_shape=jax.ShapeDtypeStruct((NQ, N_PAD, QC), jnp.float32),
    )(aggx, w1, b1r, w2, deg_parts)


def _out_call(agg, b2r, w3, b3r, deg_parts):
    return pl.pallas_call(
        _out_tc,
        grid=(N_PAD // BN,),
        in_specs=[
            pl.BlockSpec((NQ, BN, QC), lambda i: (0, i, 0)),
            pl.BlockSpec((1, DIM_H), lambda i: (0, 0)),
            pl.BlockSpec((DIM_H, DIM_OUT), lambda i: (0, 0)),
            pl.BlockSpec((1, DIM_OUT), lambda i: (0, 0)),
            pl.BlockSpec((NW, BN), lambda i: (0, i)),
        ],
        out_specs=pl.BlockSpec((BN, DIM_OUT), lambda i: (i, 0)),
        out_shape=jax.ShapeDtypeStruct((N_PAD, DIM_OUT), jnp.float32),
    )(agg, b2r, w3, b3r, deg_parts)


def kernel(x, edge_index, W1, b1, W2, b2, W3, b3):
    e = edge_index.shape[1]
    src = edge_index[0].astype(jnp.int32)
    dst = edge_index[1].astype(jnp.int32)

    em = -(-e // (NS * ESUP)) * ESUP
    pad = NS * em - e
    fill = jnp.full((pad,), N_NODES, jnp.int32)
    srcp = jnp.concatenate([src, fill]).reshape(NS, em // ESUP, KSUP, CB)
    dstp = jnp.concatenate([dst, fill]).reshape(NS, em // ESUP, KSUP, CB)
    nsup = em // ESUP

    deg_parts = _make_hist(e // NW)(dst.reshape(NW, e // NW))

    xp = jnp.pad(x, ((0, N_PAD - N_NODES), (0, 0)))
    b1r = b1.reshape(1, DIM_H)
    b2r = b2.reshape(1, DIM_H)
    b3r = b3.reshape(1, DIM_OUT)

    xs = _scale_call(xp, deg_parts)
    aggx = _make_mp(nsup, 1)(xs, srcp, dstp)
    hs2 = _mid_call(aggx, W1, b1r, W2, deg_parts)
    agg2 = _make_mp(nsup, NQ // NC)(hs2, srcp, dstp)
    return _out_call(agg2, b2r, W3, b3r, deg_parts)[:N_NODES]
